# baseline (speedup 1.0000x reference)
import functools

import jax
import jax.numpy as jnp
from jax import lax
from jax.experimental import pallas as pl
from jax.experimental.pallas import tpu as pltpu
from jax.experimental.pallas import tpu_sc as plsc

N = 10000
E = 320000
D_IN = 128
D_H = 128
D_OUT = 40
D2H = D_H // 2
D_OP = 48
D2O = D_OP // 2

NC = 2
NS = 16
CH = 256
K2 = 80
E_PAD = NS * K2 * CH
N_PAD = 10112
RPT = N_PAD // NS
NBUF = 3
LOOK = 2

_UNTILED = pltpu.CompilerParams(use_tc_tiling_on_sc=False)


def _mesh():
    return plsc.VectorSubcoreMesh(
        core_axis_name="c", subcore_axis_name="s", num_cores=NC, num_subcores=NS
    )


CHD = 128
KD = E_PAD // (NC * NS * CHD)


@functools.cache
def _make_degree():
    return functools.partial(
        pl.kernel,
        out_type=jax.ShapeDtypeStruct((NC * N_PAD,), jnp.float32),
        mesh=_mesh(),
        scratch_types=[
            pltpu.VMEM((KD, CHD), jnp.int32),
            pltpu.VMEM((CHD,), jnp.float32),
            pltpu.VMEM((RPT,), jnp.float32),
            pltpu.VMEM_SHARED((N_PAD,), jnp.float32),
            pltpu.SemaphoreType.DMA,
        ],
    )(_sc_degree_body)


def _sc_degree_body(dst_hbm, out_hbm, didx, ones_v, zbuf, acc, sem):
    c = lax.axis_index("c")
    s = lax.axis_index("s")
    wid = c * NS + s

    pltpu.sync_copy(dst_hbm.at[wid], didx)

    for i in range(CHD // 16):
        ones_v[pl.ds(i * 16, 16)] = jnp.ones((16,), jnp.float32)

    def zfill(i, _):
        zbuf[pl.ds(i * 16, 16)] = jnp.zeros((16,), jnp.float32)
        return 0

    lax.fori_loop(0, RPT // 16, zfill, 0)
    zbuf[pl.ds(RPT - 16, 16)] = jnp.zeros((16,), jnp.float32)

    pltpu.sync_copy(zbuf, acc.at[pl.ds(s * RPT, RPT)])
    plsc.subcore_barrier()

    def body(jj, _):
        descs = [
            pltpu.async_copy(ones_v, acc.at[didx.at[jj * 8 + b]], sem, add=True)
            for b in range(8)
        ]
        for d in descs:
            d.wait()
        return 0

    lax.fori_loop(0, KD // 8, body, 0)
    plsc.subcore_barrier()
    pltpu.sync_copy(acc.at[pl.ds(s * RPT, RPT)], zbuf)
    pltpu.sync_copy(zbuf, out_hbm.at[pl.ds(c * N_PAD + s * RPT, RPT)])


@functools.cache
def _make_scatter(D2):
    @functools.partial(
        pl.kernel,
        out_type=jax.ShapeDtypeStruct((NC * N_PAD, D2), jnp.float32),
        compiler_params=_UNTILED,
        mesh=_mesh(),
        scratch_types=[
            pltpu.VMEM((K2, CH), jnp.int32),
            pltpu.VMEM((K2, CH), jnp.int32),
            pltpu.VMEM((NBUF, CH, D2), jnp.float32),
            pltpu.VMEM_SHARED((N_PAD, D2), jnp.float32),
            pltpu.SemaphoreType.DMA,
            pltpu.SemaphoreType.DMA,
            pltpu.SemaphoreType.DMA,
            pltpu.SemaphoreType.DMA,
            pltpu.SemaphoreType.DMA,
            pltpu.SemaphoreType.DMA,
            pltpu.SemaphoreType.DMA,
            pltpu.SemaphoreType.DMA,
            pltpu.SemaphoreType.DMA,
            pltpu.SemaphoreType.DMA,
            pltpu.SemaphoreType.DMA,
        ],
    )
    def scatter(src_hbm, dst_hbm, h_hbm, out_hbm, sidx, didx, rows, acc,
                sem_z, sg0, sg1, sg2, sg3, sg4, ss0, ss1, ss2, ss3, ss4):
        c = lax.axis_index("c")
        s = lax.axis_index("s")
        sg = (sg0, sg1, sg2, sg3, sg4)
        ss = (ss0, ss1, ss2, ss3, ss4)

        pltpu.sync_copy(src_hbm.at[c, s], sidx)
        pltpu.sync_copy(dst_hbm.at[s], didx)

        col_offs = list(range(0, D2 - 15, 16))
        if D2 % 16:
            col_offs.append(D2 - 16)
        for r in range(8):
            for o in col_offs:
                rows[0, r, pl.ds(o, 16)] = jnp.zeros((16,), jnp.float32)

        def zbody(r, _):
            pltpu.async_copy(rows.at[0, pl.ds(0, 8)],
                             acc.at[pl.ds(s * RPT + r * 8, 8)], sem_z)
            return 0

        lax.fori_loop(0, RPT // 8, zbody, 0)

        def zdrain(r, _):
            pltpu.make_async_copy(rows.at[0, pl.ds(0, 8)],
                                  acc.at[pl.ds(s * RPT + r * 8, 8)], sem_z).wait()
            return 0

        lax.fori_loop(0, RPT // 8, zdrain, 0)
        plsc.subcore_barrier()

        def gath(j, b):
            pltpu.async_copy(h_hbm.at[sidx.at[j]], rows.at[b], sg[b])

        def gwait(j, b):
            pltpu.make_async_copy(h_hbm.at[sidx.at[j]], rows.at[b], sg[b]).wait()

        def scat(j, b):
            pltpu.async_copy(rows.at[b], acc.at[didx.at[j]], ss[b], add=True)

        def swaitf(j, b):
            pltpu.make_async_copy(rows.at[b], acc.at[didx.at[j]], ss[b]).wait()

        for m in range(LOOK):
            gath(m, m % NBUF)

        def step(j, slot, do_swait, do_gath):
            gwait(j, slot)
            scat(j, slot)
            ms = (slot + LOOK) % NBUF
            if do_swait:
                swaitf(j - (NBUF - LOOK), ms)
            if do_gath:
                gath(j + LOOK, ms)

        peel = max(NBUF - LOOK, 1)
        blocks = (K2 - LOOK - peel) // NBUF
        tail0 = peel + NBUF * blocks

        for j in range(peel):
            step(j, j % NBUF, j >= NBUF - LOOK, j + LOOK < K2)

        def body(jj, _):
            for b in range(NBUF):
                step(peel + jj * NBUF + b, (peel + b) % NBUF, True, True)
            return 0

        lax.fori_loop(0, blocks, body, 0)

        for j in range(tail0, K2):
            step(j, j % NBUF, j >= NBUF - LOOK, j + LOOK < K2)
        for j in range(K2 - (NBUF - LOOK), K2):
            swaitf(j, j % NBUF)

        plsc.subcore_barrier()
        pltpu.sync_copy(acc.at[pl.ds(s * RPT, RPT)],
                        out_hbm.at[pl.ds(c * N_PAD + s * RPT, RPT)])

    return scatter


def _tc1_body(x_ref, w_ref, degp_ref, hp_ref, s_ref):
    deg = degp_ref[0] + degp_ref[1] + 1.0
    sv = lax.rsqrt(jnp.maximum(deg, 1.0))
    s_ref[...] = sv
    h = jnp.dot(x_ref[...], w_ref[...], preferred_element_type=jnp.float32)
    hs = h * sv[:N]
    z = jnp.zeros((N_PAD - N, D2H), jnp.float32)
    hp_ref[:N] = hs[:, :D2H]
    hp_ref[N:N_PAD] = z
    hp_ref[N_PAD:N_PAD + N] = hs[:, D2H:]
    hp_ref[N_PAD + N:] = z


def _tc1(x, W1, degp):
    return pl.pallas_call(
        _tc1_body,
        out_shape=(
            jax.ShapeDtypeStruct((NC * N_PAD, D2H), jnp.float32),
            jax.ShapeDtypeStruct((N_PAD, 1), jnp.float32),
        ),
    )(x, W1, degp)


def _tc2_body(aggp_ref, hp_ref, s_ref, b1_ref, g1_ref, be1_ref, w2_ref, h2p_ref):
    sN = s_ref[:N]
    left = aggp_ref[:N] + hp_ref[:N]
    right = aggp_ref[N_PAD:N_PAD + N] + hp_ref[N_PAD:N_PAD + N]
    a = sN * jnp.concatenate([left, right], axis=1) + b1_ref[...]
    mean = jnp.mean(a, axis=0, keepdims=True)
    cen = a - mean
    var = jnp.mean(cen * cen, axis=0, keepdims=True)
    h1 = jnp.maximum(g1_ref[...] * cen * lax.rsqrt(var + 1e-5) + be1_ref[...], 0.0)
    h2 = jnp.dot(h1, w2_ref[...], preferred_element_type=jnp.float32)
    h2s = h2 * sN
    z = jnp.zeros((N_PAD - N, D2O), jnp.float32)
    h2p_ref[:N] = h2s[:, :D2O]
    h2p_ref[N:N_PAD] = z
    h2p_ref[N_PAD:N_PAD + N] = h2s[:, D2O:]
    h2p_ref[N_PAD + N:] = z


def _tc2(aggp, hp, s, b1, g1, be1, W2):
    return pl.pallas_call(
        _tc2_body,
        out_shape=jax.ShapeDtypeStruct((NC * N_PAD, D2O), jnp.float32),
    )(aggp, hp, s, b1, g1, be1, W2)


def _tc3_body(agg2p_ref, h2p_ref, s_ref, b2_ref, out_ref):
    left = agg2p_ref[:N] + h2p_ref[:N]
    right = agg2p_ref[N_PAD:N_PAD + N] + h2p_ref[N_PAD:N_PAD + N]
    o = (s_ref[:N] * jnp.concatenate([left, right], axis=1))[:, :D_OUT] + b2_ref[...]
    m = jnp.max(o, axis=1, keepdims=True)
    lse = jnp.log(jnp.sum(jnp.exp(o - m), axis=1, keepdims=True))
    out_ref[...] = o - m - lse


def _tc3(agg2p, h2p, s, b2):
    return pl.pallas_call(
        _tc3_body,
        out_shape=jax.ShapeDtypeStruct((N, D_OUT), jnp.float32),
    )(agg2p, h2p, s, b2)


def kernel(x, W1, b1, gamma1, beta1, W2, b2, edge_index):
    pad = jnp.full((E_PAD - E,), N, jnp.int32)
    src = jnp.concatenate([edge_index[0], pad])
    dst = jnp.concatenate([edge_index[1], pad])
    dstp = dst.reshape(NS, K2, CH)
    dstp_deg = dst.reshape(NC * NS, KD, CHD)
    srcp = jnp.stack([src, src + N_PAD]).reshape(NC, NS, K2, CH)

    degp = _make_degree()(dstp_deg).reshape(NC, N_PAD, 1)
    hp, s = _tc1(x, W1, degp)
    aggp = _make_scatter(D2H)(srcp, dstp, hp)
    W2p = jnp.pad(W2, ((0, 0), (0, D_OP - D_OUT)))
    h2p = _tc2(aggp, hp, s, b1.reshape(1, D_H), gamma1.reshape(1, D_H),
               beta1.reshape(1, D_H), W2p)
    agg2p = _make_scatter(D2O)(srcp, dstp, h2p)
    return _tc3(agg2p, h2p, s, b2.reshape(1, D_OUT))


# speedup vs baseline: 1.8373x; 1.8373x over previous
import functools

import jax
import jax.numpy as jnp
from jax import lax
from jax.experimental import pallas as pl
from jax.experimental.pallas import tpu as pltpu
from jax.experimental.pallas import tpu_sc as plsc

N = 10000
E = 320000
D_IN = 128
D_H = 128
D_OUT = 40
D2H = D_H // 2
D_OP = 48
D2O = D_OP // 2

NC = 2
NS = 16
EPT = E // NS
CH = 160
K2 = EPT // CH
N_PAD = 10112
RPT = N_PAD // NS
NBUF = 4
LOOK = 3
E_PAD = NC * NS * 80 * 128

_UNTILED = pltpu.CompilerParams(use_tc_tiling_on_sc=False)


def _mesh():
    return plsc.VectorSubcoreMesh(
        core_axis_name="c", subcore_axis_name="s", num_cores=NC, num_subcores=NS
    )


CHD = 128
KD = E_PAD // (NC * NS * CHD)


@functools.cache
def _make_degree():
    return functools.partial(
        pl.kernel,
        out_type=jax.ShapeDtypeStruct((NC * N_PAD,), jnp.float32),
        mesh=_mesh(),
        scratch_types=[
            pltpu.VMEM((KD, CHD), jnp.int32),
            pltpu.VMEM((CHD,), jnp.float32),
            pltpu.VMEM((RPT,), jnp.float32),
            pltpu.VMEM_SHARED((N_PAD,), jnp.float32),
            pltpu.SemaphoreType.DMA,
        ],
    )(_sc_degree_body)


def _sc_degree_body(dst_hbm, out_hbm, didx, ones_v, zbuf, acc, sem):
    c = lax.axis_index("c")
    s = lax.axis_index("s")
    wid = c * NS + s

    pltpu.sync_copy(dst_hbm.at[wid], didx)

    for i in range(CHD // 16):
        ones_v[pl.ds(i * 16, 16)] = jnp.ones((16,), jnp.float32)

    def zfill(i, _):
        zbuf[pl.ds(i * 16, 16)] = jnp.zeros((16,), jnp.float32)
        return 0

    lax.fori_loop(0, RPT // 16, zfill, 0)
    zbuf[pl.ds(RPT - 16, 16)] = jnp.zeros((16,), jnp.float32)

    pltpu.sync_copy(zbuf, acc.at[pl.ds(s * RPT, RPT)])
    plsc.subcore_barrier()

    def body(jj, _):
        descs = [
            pltpu.async_copy(ones_v, acc.at[didx.at[jj * 8 + b]], sem, add=True)
            for b in range(8)
        ]
        for d in descs:
            d.wait()
        return 0

    lax.fori_loop(0, KD // 8, body, 0)
    plsc.subcore_barrier()
    pltpu.sync_copy(acc.at[pl.ds(s * RPT, RPT)], zbuf)
    pltpu.sync_copy(zbuf, out_hbm.at[pl.ds(c * N_PAD + s * RPT, RPT)])


@functools.cache
def _make_scatter(D2):
    @functools.partial(
        pl.kernel,
        out_type=jax.ShapeDtypeStruct((NC * N_PAD, D2), jnp.float32),
        compiler_params=_UNTILED,
        mesh=_mesh(),
        scratch_types=[
            pltpu.VMEM((EPT,), jnp.int32),
            pltpu.VMEM((EPT,), jnp.int32),
            pltpu.VMEM((NBUF, CH, D2), jnp.float32),
            pltpu.VMEM_SHARED((N_PAD, D2), jnp.float32),
            pltpu.SemaphoreType.DMA,
            pltpu.SemaphoreType.DMA,
            pltpu.SemaphoreType.DMA,
            pltpu.SemaphoreType.DMA,
            pltpu.SemaphoreType.DMA,
            pltpu.SemaphoreType.DMA,
            pltpu.SemaphoreType.DMA,
            pltpu.SemaphoreType.DMA,
            pltpu.SemaphoreType.DMA,
            pltpu.SemaphoreType.DMA,
            pltpu.SemaphoreType.DMA,
        ],
    )
    def scatter(ei_hbm, h_hbm, out_hbm, sidx, didx, rows, acc,
                sem_z, sg0, sg1, sg2, sg3, sg4, ss0, ss1, ss2, ss3, ss4):
        c = lax.axis_index("c")
        s = lax.axis_index("s")
        sg = (sg0, sg1, sg2, sg3, sg4)
        ss = (ss0, ss1, ss2, ss3, ss4)

        pltpu.sync_copy(ei_hbm.at[0, pl.ds(s * EPT, EPT)], sidx)
        pltpu.sync_copy(ei_hbm.at[1, pl.ds(s * EPT, EPT)], didx)

        off = c * N_PAD

        def obody(i, _):
            sidx[pl.ds(i * 16, 16)] = sidx[pl.ds(i * 16, 16)] + off
            return 0

        lax.fori_loop(0, EPT // 16, obody, 0)

        col_offs = list(range(0, D2 - 15, 16))
        if D2 % 16:
            col_offs.append(D2 - 16)
        for r in range(8):
            for o in col_offs:
                rows[0, r, pl.ds(o, 16)] = jnp.zeros((16,), jnp.float32)

        def zbody(r, _):
            pltpu.async_copy(rows.at[0, pl.ds(0, 8)],
                             acc.at[pl.ds(s * RPT + r * 8, 8)], sem_z)
            return 0

        lax.fori_loop(0, RPT // 8, zbody, 0)

        def zdrain(r, _):
            pltpu.make_async_copy(rows.at[0, pl.ds(0, 8)],
                                  acc.at[pl.ds(s * RPT + r * 8, 8)], sem_z).wait()
            return 0

        lax.fori_loop(0, RPT // 8, zdrain, 0)
        plsc.subcore_barrier()

        def gath(j, b):
            pltpu.async_copy(h_hbm.at[sidx.at[pl.ds(j * CH, CH)]], rows.at[b], sg[b])

        def gwait(j, b):
            pltpu.make_async_copy(
                h_hbm.at[sidx.at[pl.ds(j * CH, CH)]], rows.at[b], sg[b]).wait()

        def scat(j, b):
            pltpu.async_copy(rows.at[b], acc.at[didx.at[pl.ds(j * CH, CH)]],
                             ss[b], add=True)

        def swaitf(j, b):
            pltpu.make_async_copy(
                rows.at[b], acc.at[didx.at[pl.ds(j * CH, CH)]], ss[b]).wait()

        for m in range(LOOK):
            gath(m, m % NBUF)

        def step(j, slot, do_swait, do_gath):
            gwait(j, slot)
            scat(j, slot)
            ms = (slot + LOOK) % NBUF
            if do_swait:
                swaitf(j - (NBUF - LOOK), ms)
            if do_gath:
                gath(j + LOOK, ms)

        peel = max(NBUF - LOOK, 1)
        blocks = (K2 - LOOK - peel) // NBUF
        tail0 = peel + NBUF * blocks

        for j in range(peel):
            step(j, j % NBUF, j >= NBUF - LOOK, j + LOOK < K2)

        def body(jj, _):
            for b in range(NBUF):
                step(peel + jj * NBUF + b, (peel + b) % NBUF, True, True)
            return 0

        lax.fori_loop(0, blocks, body, 0)

        for j in range(tail0, K2):
            step(j, j % NBUF, j >= NBUF - LOOK, j + LOOK < K2)
        for j in range(K2 - (NBUF - LOOK), K2):
            swaitf(j, j % NBUF)

        plsc.subcore_barrier()
        pltpu.sync_copy(acc.at[pl.ds(s * RPT, RPT)],
                        out_hbm.at[pl.ds(c * N_PAD + s * RPT, RPT)])

    return scatter


def _tc1_body(x_ref, w_ref, degp_ref, hp_ref, s_ref):
    deg = degp_ref[0] + degp_ref[1] + 1.0
    sv = lax.rsqrt(jnp.maximum(deg, 1.0))
    s_ref[...] = sv
    h = jnp.dot(x_ref[...], w_ref[...], preferred_element_type=jnp.float32)
    hs = h * sv[:N]
    z = jnp.zeros((N_PAD - N, D2H), jnp.float32)
    hp_ref[:N] = hs[:, :D2H]
    hp_ref[N:N_PAD] = z
    hp_ref[N_PAD:N_PAD + N] = hs[:, D2H:]
    hp_ref[N_PAD + N:] = z


def _tc1(x, W1, degp):
    return pl.pallas_call(
        _tc1_body,
        out_shape=(
            jax.ShapeDtypeStruct((NC * N_PAD, D2H), jnp.float32),
            jax.ShapeDtypeStruct((N_PAD, 1), jnp.float32),
        ),
    )(x, W1, degp)


def _tc2_body(aggp_ref, hp_ref, s_ref, b1_ref, g1_ref, be1_ref, w2_ref, h2p_ref):
    sN = s_ref[:N]
    left = aggp_ref[:N] + hp_ref[:N]
    right = aggp_ref[N_PAD:N_PAD + N] + hp_ref[N_PAD:N_PAD + N]
    a = sN * jnp.concatenate([left, right], axis=1) + b1_ref[...]
    mean = jnp.mean(a, axis=0, keepdims=True)
    cen = a - mean
    var = jnp.mean(cen * cen, axis=0, keepdims=True)
    h1 = jnp.maximum(g1_ref[...] * cen * lax.rsqrt(var + 1e-5) + be1_ref[...], 0.0)
    h2 = jnp.dot(h1, w2_ref[...], preferred_element_type=jnp.float32)
    h2s = h2 * sN
    z = jnp.zeros((N_PAD - N, D2O), jnp.float32)
    h2p_ref[:N] = h2s[:, :D2O]
    h2p_ref[N:N_PAD] = z
    h2p_ref[N_PAD:N_PAD + N] = h2s[:, D2O:]
    h2p_ref[N_PAD + N:] = z


def _tc2(aggp, hp, s, b1, g1, be1, W2):
    return pl.pallas_call(
        _tc2_body,
        out_shape=jax.ShapeDtypeStruct((NC * N_PAD, D2O), jnp.float32),
    )(aggp, hp, s, b1, g1, be1, W2)


def _tc3_body(agg2p_ref, h2p_ref, s_ref, b2_ref, out_ref):
    left = agg2p_ref[:N] + h2p_ref[:N]
    right = agg2p_ref[N_PAD:N_PAD + N] + h2p_ref[N_PAD:N_PAD + N]
    o = (s_ref[:N] * jnp.concatenate([left, right], axis=1))[:, :D_OUT] + b2_ref[...]
    m = jnp.max(o, axis=1, keepdims=True)
    lse = jnp.log(jnp.sum(jnp.exp(o - m), axis=1, keepdims=True))
    out_ref[...] = o - m - lse


def _tc3(agg2p, h2p, s, b2):
    return pl.pallas_call(
        _tc3_body,
        out_shape=jax.ShapeDtypeStruct((N, D_OUT), jnp.float32),
    )(agg2p, h2p, s, b2)


def kernel(x, W1, b1, gamma1, beta1, W2, b2, edge_index):
    pad = jnp.full((E_PAD - E,), N, jnp.int32)
    dstp_deg = jnp.concatenate([edge_index[1], pad]).reshape(NC * NS, KD, CHD)

    degp = _make_degree()(dstp_deg).reshape(NC, N_PAD, 1)
    hp, s = _tc1(x, W1, degp)
    aggp = _make_scatter(D2H)(edge_index, hp)
    W2p = jnp.pad(W2, ((0, 0), (0, D_OP - D_OUT)))
    h2p = _tc2(aggp, hp, s, b1.reshape(1, D_H), gamma1.reshape(1, D_H),
               beta1.reshape(1, D_H), W2p)
    agg2p = _make_scatter(D2O)(edge_index, h2p)
    return _tc3(agg2p, h2p, s, b2.reshape(1, D_OUT))
